# baseline (speedup 1.0000x reference)
import functools

import jax
import jax.numpy as jnp
import numpy as np
from jax.experimental import pallas as pl
from jax.experimental.pallas import tpu as pltpu

_HIGH = jax.lax.Precision.HIGHEST


def _proj_kernel(x_ref, w_ref, b_ref, out_ref):
    out_ref[0] = (
        jax.lax.dot_general(
            x_ref[...], w_ref[0], (((1,), (1,)), ((), ())),
            preferred_element_type=jnp.float32, precision=_HIGH)
        + b_ref[0]
    )


def _head_kernel(u, L, qkv_ref, wo_ref, onehot_ref, contrib_ref, *, scale):
    q = qkv_ref[0]
    k = qkv_ref[1]
    v = qkv_ref[2]
    scores_t = jax.lax.dot_general(
        k, q, (((1,), (1,)), ((), ())),
        preferred_element_type=jnp.float32, precision=_HIGH) * scale
    row_max = jnp.max(scores_t, axis=0, keepdims=True)
    iota = jax.lax.broadcasted_iota(jnp.int32, (1, L), 1)
    r = row_max
    oh_rows = []
    for _ in range(u):
        m = jnp.max(r, axis=1, keepdims=True)
        idx = jnp.min(jnp.where(r >= m, iota, L), axis=1, keepdims=True)
        oh = iota == idx
        oh_rows.append(oh.astype(jnp.float32))
        r = jnp.where(oh, -jnp.inf, r)
    up = onehot_ref.shape[1]
    for _ in range(up - u):
        oh_rows.append(jnp.zeros((1, L), jnp.float32))
    onehot = jnp.concatenate(oh_rows, axis=0)
    onehot_ref[0] = onehot
    attn_sel = jax.lax.dot_general(
        onehot, scores_t, (((1,), (1,)), ((), ())),
        preferred_element_type=jnp.float32, precision=_HIGH)
    mx = jnp.max(attn_sel, axis=1, keepdims=True)
    e = jnp.exp(attn_sel - mx)
    w = e / jnp.sum(e, axis=1, keepdims=True)
    ctx = jax.lax.dot_general(
        w, v, (((1,), (0,)), ((), ())),
        preferred_element_type=jnp.float32, precision=_HIGH)
    contrib_ref[0] = jax.lax.dot_general(
        ctx, wo_ref[...], (((1,), (1,)), ((), ())),
        preferred_element_type=jnp.float32, precision=_HIGH)


def _scatter_kernel(oh_ref, contrib_ref, bo_ref, out_ref):
    out_ref[...] = (
        jax.lax.dot_general(
            oh_ref[...], contrib_ref[...], (((0,), (0,)), ((), ())),
            preferred_element_type=jnp.float32, precision=_HIGH)
        + bo_ref[...]
    )


def kernel(x, Wq, bq, Wk, bk, Wv, bv, Wo, bo):
    B, L, d = x.shape
    H = 16
    dk = d // H
    u = min(L, max(1, int(np.log(float(L)))))
    up = max(8, -(-u // 8) * 8)
    scale = 1.0 / np.sqrt(dk)
    x2 = x.reshape(L, d)
    Ws = jnp.stack([Wq, Wk, Wv])
    bs = jnp.stack([bq, bk, bv])[:, None, :]

    blk = 256
    qkv = pl.pallas_call(
        _proj_kernel,
        grid=(3, L // blk),
        in_specs=[
            pl.BlockSpec((blk, d), lambda i, j: (j, 0)),
            pl.BlockSpec((1, d, d), lambda i, j: (i, 0, 0)),
            pl.BlockSpec((1, 1, d), lambda i, j: (i, 0, 0)),
        ],
        out_specs=pl.BlockSpec((1, blk, d), lambda i, j: (i, j, 0)),
        out_shape=jax.ShapeDtypeStruct((3, L, d), jnp.float32),
    )(x2, Ws, bs)

    onehot, contrib = pl.pallas_call(
        functools.partial(_head_kernel, u, L, scale=scale),
        grid=(H,),
        in_specs=[
            pl.BlockSpec((3, L, dk), lambda h: (0, 0, h)),
            pl.BlockSpec((d, dk), lambda h: (0, h)),
        ],
        out_specs=[
            pl.BlockSpec((1, up, L), lambda h: (h, 0, 0)),
            pl.BlockSpec((1, up, d), lambda h: (h, 0, 0)),
        ],
        out_shape=[
            jax.ShapeDtypeStruct((H, up, L), jnp.float32),
            jax.ShapeDtypeStruct((H, up, d), jnp.float32),
        ],
    )(qkv, Wo)

    oh_flat = onehot.reshape(H * up, L)
    contrib_flat = contrib.reshape(H * up, d)

    out = pl.pallas_call(
        _scatter_kernel,
        grid=(L // blk,),
        in_specs=[
            pl.BlockSpec((H * up, blk), lambda j: (0, j)),
            pl.BlockSpec((H * up, d), lambda j: (0, 0)),
            pl.BlockSpec((1, d), lambda j: (0, 0)),
        ],
        out_specs=pl.BlockSpec((blk, d), lambda j: (j, 0)),
        out_shape=jax.ShapeDtypeStruct((L, d), jnp.float32),
    )(oh_flat, contrib_flat, bo[None, :])

    return out.reshape(B, L, d)


# speedup vs baseline: 1.2426x; 1.2426x over previous
import functools

import jax
import jax.numpy as jnp
import numpy as np
from jax.experimental import pallas as pl
from jax.experimental.pallas import tpu as pltpu

_HIGH = jax.lax.Precision.HIGHEST


def _proj_kernel(x_ref, wt_ref, b_ref, out_ref):
    out_ref[0] = (
        jax.lax.dot_general(
            x_ref[...], wt_ref[0], (((1,), (0,)), ((), ())),
            preferred_element_type=jnp.float32, precision=_HIGH)
        + b_ref[0]
    )


def _head_kernel(u, L, q_ref, k_ref, v_ref, wot_ref, onehot_ref, contrib_ref,
                 *, scale):
    qb = q_ref[0].astype(jnp.bfloat16)
    kb = k_ref[0].astype(jnp.bfloat16)
    scores = jax.lax.dot_general(
        qb, kb, (((1,), (1,)), ((), ())),
        preferred_element_type=jnp.float32) * scale
    row_max = jnp.max(scores, axis=1, keepdims=True)
    iota = jax.lax.broadcasted_iota(jnp.int32, (L, 1), 0)
    r = row_max
    oh_cols = []
    for _ in range(u):
        m = jnp.max(r, axis=0, keepdims=True)
        idx = jnp.min(jnp.where(r >= m, iota, L), axis=0, keepdims=True)
        oh = iota == idx
        oh_cols.append(oh.astype(jnp.float32))
        r = jnp.where(oh, -jnp.inf, r)
    up = onehot_ref.shape[2]
    for _ in range(up - u):
        oh_cols.append(jnp.zeros((L, 1), jnp.float32))
    onehot_t = jnp.concatenate(oh_cols, axis=1)
    onehot_ref[0] = onehot_t
    attn_sel = jax.lax.dot_general(
        onehot_t, scores, (((0,), (0,)), ((), ())),
        preferred_element_type=jnp.float32, precision=_HIGH)
    mx = jnp.max(attn_sel, axis=1, keepdims=True)
    e = jnp.exp(attn_sel - mx)
    w = e / jnp.sum(e, axis=1, keepdims=True)
    ctx = jax.lax.dot_general(
        w.astype(jnp.bfloat16), v_ref[0].astype(jnp.bfloat16),
        (((1,), (0,)), ((), ())),
        preferred_element_type=jnp.float32)
    contrib_ref[0] = jax.lax.dot_general(
        ctx, wot_ref[...], (((1,), (0,)), ((), ())),
        preferred_element_type=jnp.float32, precision=_HIGH)


def _scatter_kernel(oh_ref, contrib_ref, bo_ref, out_ref):
    out_ref[...] = (
        jax.lax.dot_general(
            oh_ref[...], contrib_ref[...], (((1,), (0,)), ((), ())),
            preferred_element_type=jnp.float32, precision=_HIGH)
        + bo_ref[...]
    )


def kernel(x, Wq, bq, Wk, bk, Wv, bv, Wo, bo):
    B, L, d = x.shape
    H = 16
    dk = d // H
    u = min(L, max(1, int(np.log(float(L)))))
    up = max(8, -(-u // 8) * 8)
    scale = 1.0 / np.sqrt(dk)
    x2 = x.reshape(L, d)
    Wts = jnp.stack([Wq.T, Wk.T, Wv.T])
    bs = jnp.stack([bq, bk, bv])[:, None, :]
    WoT = Wo.T

    blk = 256
    qkv = pl.pallas_call(
        _proj_kernel,
        grid=(3, L // blk),
        in_specs=[
            pl.BlockSpec((blk, d), lambda i, j: (j, 0)),
            pl.BlockSpec((1, d, d), lambda i, j: (i, 0, 0)),
            pl.BlockSpec((1, 1, d), lambda i, j: (i, 0, 0)),
        ],
        out_specs=pl.BlockSpec((1, blk, d), lambda i, j: (i, j, 0)),
        out_shape=jax.ShapeDtypeStruct((3, L, d), jnp.float32),
    )(x2, Wts, bs)

    onehot_t, contrib = pl.pallas_call(
        functools.partial(_head_kernel, u, L, scale=scale),
        grid=(H,),
        in_specs=[
            pl.BlockSpec((1, L, dk), lambda h: (0, 0, h)),
            pl.BlockSpec((1, L, dk), lambda h: (1, 0, h)),
            pl.BlockSpec((1, L, dk), lambda h: (2, 0, h)),
            pl.BlockSpec((dk, d), lambda h: (h, 0)),
        ],
        out_specs=[
            pl.BlockSpec((1, L, up), lambda h: (h, 0, 0)),
            pl.BlockSpec((1, up, d), lambda h: (h, 0, 0)),
        ],
        out_shape=[
            jax.ShapeDtypeStruct((H, L, up), jnp.float32),
            jax.ShapeDtypeStruct((H, up, d), jnp.float32),
        ],
    )(qkv, qkv, qkv, WoT)

    oh_cat = jnp.concatenate(list(onehot_t), axis=1)
    contrib_flat = contrib.reshape(H * up, d)

    out = pl.pallas_call(
        _scatter_kernel,
        grid=(L // blk,),
        in_specs=[
            pl.BlockSpec((blk, H * up), lambda j: (j, 0)),
            pl.BlockSpec((H * up, d), lambda j: (0, 0)),
            pl.BlockSpec((1, d), lambda j: (0, 0)),
        ],
        out_specs=pl.BlockSpec((blk, d), lambda j: (j, 0)),
        out_shape=jax.ShapeDtypeStruct((L, d), jnp.float32),
    )(oh_cat, contrib_flat, bo[None, :])

    return out.reshape(B, L, d)


# speedup vs baseline: 1.4142x; 1.1381x over previous
import functools

import jax
import jax.numpy as jnp
import numpy as np
from jax.experimental import pallas as pl
from jax.experimental.pallas import tpu as pltpu

_HIGH = jax.lax.Precision.HIGHEST


def _proj_kernel(x_ref, w_ref, b_ref, out_ref):
    out_ref[...] = (
        jax.lax.dot_general(
            x_ref[...], w_ref[...], (((1,), (1,)), ((), ())),
            preferred_element_type=jnp.float32, precision=_HIGH)
        + b_ref[...]
    )


def _head_kernel(u, L, q_ref, k_ref, v_ref, wo_ref, onehot_ref, contrib_ref,
                 *, scale):
    qb = q_ref[...].astype(jnp.bfloat16)
    kb = k_ref[...].astype(jnp.bfloat16)
    scores = jax.lax.dot_general(
        qb, kb, (((1,), (1,)), ((), ())),
        preferred_element_type=jnp.float32) * scale
    row_max = jnp.max(scores, axis=1, keepdims=True)
    iota = jax.lax.broadcasted_iota(jnp.int32, (L, 1), 0)
    r = row_max
    oh_cols = []
    for _ in range(u):
        m = jnp.max(r, axis=0, keepdims=True)
        idx = jnp.min(jnp.where(r >= m, iota, L), axis=0, keepdims=True)
        oh = iota == idx
        oh_cols.append(oh.astype(jnp.float32))
        r = jnp.where(oh, -jnp.inf, r)
    up = onehot_ref.shape[2]
    for _ in range(up - u):
        oh_cols.append(jnp.zeros((L, 1), jnp.float32))
    onehot_t = jnp.concatenate(oh_cols, axis=1)
    onehot_ref[0] = onehot_t
    attn_sel = jax.lax.dot_general(
        onehot_t, scores, (((0,), (0,)), ((), ())),
        preferred_element_type=jnp.float32, precision=_HIGH)
    mx = jnp.max(attn_sel, axis=1, keepdims=True)
    e = jnp.exp(attn_sel - mx)
    w = e / jnp.sum(e, axis=1, keepdims=True)
    ctx = jax.lax.dot_general(
        w.astype(jnp.bfloat16), v_ref[...].astype(jnp.bfloat16),
        (((1,), (0,)), ((), ())),
        preferred_element_type=jnp.float32)
    contrib_ref[...] = jax.lax.dot_general(
        ctx, wo_ref[...], (((1,), (1,)), ((), ())),
        preferred_element_type=jnp.float32, precision=_HIGH)


def _scatter_kernel(oh_ref, contrib_ref, bo_ref, out_ref):
    out_ref[...] = (
        jax.lax.dot_general(
            oh_ref[...], contrib_ref[...], (((1,), (0,)), ((), ())),
            preferred_element_type=jnp.float32, precision=_HIGH)
        + bo_ref[...]
    )


def kernel(x, Wq, bq, Wk, bk, Wv, bv, Wo, bo):
    B, L, d = x.shape
    H = 16
    dk = d // H
    u = min(L, max(1, int(np.log(float(L)))))
    up = max(8, -(-u // 8) * 8)
    scale = 1.0 / np.sqrt(dk)
    x2 = x.reshape(L, d)

    blk = 256
    proj = pl.pallas_call(
        _proj_kernel,
        grid=(L // blk,),
        in_specs=[
            pl.BlockSpec((blk, d), lambda j: (j, 0)),
            pl.BlockSpec((d, d), lambda j: (0, 0)),
            pl.BlockSpec((1, d), lambda j: (0, 0)),
        ],
        out_specs=pl.BlockSpec((blk, d), lambda j: (j, 0)),
        out_shape=jax.ShapeDtypeStruct((L, d), jnp.float32),
    )
    Q = proj(x2, Wq, bq[None, :])
    K = proj(x2, Wk, bk[None, :])
    V = proj(x2, Wv, bv[None, :])

    onehot_t, contrib = pl.pallas_call(
        functools.partial(_head_kernel, u, L, scale=scale),
        grid=(H,),
        in_specs=[
            pl.BlockSpec((L, dk), lambda h: (0, h)),
            pl.BlockSpec((L, dk), lambda h: (0, h)),
            pl.BlockSpec((L, dk), lambda h: (0, h)),
            pl.BlockSpec((d, dk), lambda h: (0, h)),
        ],
        out_specs=[
            pl.BlockSpec((1, L, up), lambda h: (h, 0, 0)),
            pl.BlockSpec((up, d), lambda h: (h, 0)),
        ],
        out_shape=[
            jax.ShapeDtypeStruct((H, L, up), jnp.float32),
            jax.ShapeDtypeStruct((H * up, d), jnp.float32),
        ],
    )(Q, K, V, Wo)

    oh_cat = jnp.concatenate(list(onehot_t), axis=1)

    out = pl.pallas_call(
        _scatter_kernel,
        grid=(L // blk,),
        in_specs=[
            pl.BlockSpec((blk, H * up), lambda j: (j, 0)),
            pl.BlockSpec((H * up, d), lambda j: (0, 0)),
            pl.BlockSpec((1, d), lambda j: (0, 0)),
        ],
        out_specs=pl.BlockSpec((blk, d), lambda j: (j, 0)),
        out_shape=jax.ShapeDtypeStruct((L, d), jnp.float32),
    )(oh_cat, contrib, bo[None, :])

    return out.reshape(B, L, d)


# speedup vs baseline: 2.2877x; 1.6177x over previous
import functools

import jax
import jax.numpy as jnp
import numpy as np
from jax.experimental import pallas as pl
from jax.experimental.pallas import tpu as pltpu

_HIGH = jax.lax.Precision.HIGHEST


def _proj_kernel(x_ref, w_ref, b_ref, out_ref, *, prec):
    out_ref[...] = (
        jax.lax.dot_general(
            x_ref[...], w_ref[...], (((1,), (1,)), ((), ())),
            preferred_element_type=jnp.float32, precision=prec)
        + b_ref[...]
    )


def _head_kernel(u, L, q_ref, k_ref, v_ref, wo_ref, onehot_ref, contrib_ref,
                 *, scale):
    qb = q_ref[...].astype(jnp.bfloat16)
    kb = k_ref[...].astype(jnp.bfloat16)
    scores = jax.lax.dot_general(
        qb, kb, (((1,), (1,)), ((), ())),
        preferred_element_type=jnp.float32)
    row_max = jnp.max(scores, axis=1, keepdims=True)
    rm = row_max.reshape(1, L)
    iota = jax.lax.broadcasted_iota(jnp.int32, (1, L), 1)
    oh_rows = []
    for _ in range(u):
        m = jnp.max(rm, axis=1, keepdims=True)
        idx = jnp.min(jnp.where(rm >= m, iota, L), axis=1, keepdims=True)
        oh = iota == idx
        oh_rows.append(oh.astype(jnp.float32))
        rm = jnp.where(oh, -jnp.inf, rm)
    up = onehot_ref.shape[0]
    for _ in range(up - u):
        oh_rows.append(jnp.zeros((1, L), jnp.float32))
    onehot = jnp.concatenate(oh_rows, axis=0)
    onehot_ref[...] = onehot
    q_sel = jax.lax.dot_general(
        onehot, q_ref[...], (((1,), (0,)), ((), ())),
        preferred_element_type=jnp.float32, precision=_HIGH)
    attn = jax.lax.dot_general(
        q_sel.astype(jnp.bfloat16), kb, (((1,), (1,)), ((), ())),
        preferred_element_type=jnp.float32) * scale
    mx = jnp.max(attn, axis=1, keepdims=True)
    e = jnp.exp(attn - mx)
    w = e / jnp.sum(e, axis=1, keepdims=True)
    ctx = jax.lax.dot_general(
        w.astype(jnp.bfloat16), v_ref[...].astype(jnp.bfloat16),
        (((1,), (0,)), ((), ())),
        preferred_element_type=jnp.float32)
    contrib_ref[...] = jax.lax.dot_general(
        ctx, wo_ref[...], (((1,), (1,)), ((), ())),
        preferred_element_type=jnp.float32, precision=_HIGH)


def _scatter_kernel(oh_ref, contrib_ref, bo_ref, out_ref):
    out_ref[...] = (
        jax.lax.dot_general(
            oh_ref[...], contrib_ref[...], (((0,), (0,)), ((), ())),
            preferred_element_type=jnp.float32, precision=_HIGH)
        + bo_ref[...]
    )


def kernel(x, Wq, bq, Wk, bk, Wv, bv, Wo, bo):
    B, L, d = x.shape
    H = 16
    dk = d // H
    u = min(L, max(1, int(np.log(float(L)))))
    up = max(8, -(-u // 8) * 8)
    scale = 1.0 / np.sqrt(dk)
    x2 = x.reshape(L, d)

    blk = 256

    def proj(W, b, prec):
        return pl.pallas_call(
            functools.partial(_proj_kernel, prec=prec),
            grid=(L // blk,),
            in_specs=[
                pl.BlockSpec((blk, d), lambda j: (j, 0)),
                pl.BlockSpec((d, d), lambda j: (0, 0)),
                pl.BlockSpec((1, d), lambda j: (0, 0)),
            ],
            out_specs=pl.BlockSpec((blk, d), lambda j: (j, 0)),
            out_shape=jax.ShapeDtypeStruct((L, d), jnp.float32),
        )(x2, W, b[None, :])

    Q = proj(Wq, bq, _HIGH)
    K = proj(Wk, bk, _HIGH)
    V = proj(Wv, bv, None)

    onehot, contrib = pl.pallas_call(
        functools.partial(_head_kernel, u, L, scale=scale),
        grid=(H,),
        in_specs=[
            pl.BlockSpec((L, dk), lambda h: (0, h)),
            pl.BlockSpec((L, dk), lambda h: (0, h)),
            pl.BlockSpec((L, dk), lambda h: (0, h)),
            pl.BlockSpec((d, dk), lambda h: (0, h)),
        ],
        out_specs=[
            pl.BlockSpec((up, L), lambda h: (h, 0)),
            pl.BlockSpec((up, d), lambda h: (h, 0)),
        ],
        out_shape=[
            jax.ShapeDtypeStruct((H * up, L), jnp.float32),
            jax.ShapeDtypeStruct((H * up, d), jnp.float32),
        ],
    )(Q, K, V, Wo)

    out = pl.pallas_call(
        _scatter_kernel,
        grid=(L // blk,),
        in_specs=[
            pl.BlockSpec((H * up, blk), lambda j: (0, j)),
            pl.BlockSpec((H * up, d), lambda j: (0, 0)),
            pl.BlockSpec((1, d), lambda j: (0, 0)),
        ],
        out_specs=pl.BlockSpec((blk, d), lambda j: (j, 0)),
        out_shape=jax.ShapeDtypeStruct((L, d), jnp.float32),
    )(onehot, contrib, bo[None, :])

    return out.reshape(B, L, d)
